# Initial kernel scaffold; baseline (speedup 1.0000x reference)
import jax
import jax.numpy as jnp
from jax.experimental import pallas as pl


def kernel(x, edge_index, Wg, bg, W1, b1, W2, b2, W3, b3):
    raise NotImplementedError("write your pallas kernel here")


# speedup vs baseline: 15.9701x; 15.9701x over previous
import functools

import jax
import jax.numpy as jnp
from jax import lax
from jax.experimental import pallas as pl
from jax.experimental.pallas import tpu as pltpu
from jax.experimental.pallas import tpu_sc as plsc

NC, NS = 2, 16
NW = NC * NS
CHUNK = 80

_MESH = dict(core_axis_name="c", subcore_axis_name="s")


def _make_deg_kernel(E, NP):
    EPT = E // NW
    NITER = EPT // CHUNK
    PPT = NP // NS

    @functools.partial(
        pl.kernel,
        out_type=jax.ShapeDtypeStruct((NC, NP), jnp.float32),
        mesh=plsc.VectorSubcoreMesh(**_MESH),
        scratch_types=[
            pltpu.VMEM((CHUNK,), jnp.int32),
            pltpu.VMEM((CHUNK,), jnp.float32),
            pltpu.VMEM((PPT,), jnp.float32),
            pltpu.VMEM_SHARED((NP,), jnp.float32),
        ],
    )
    def deg_kernel(dst_hbm, out_hbm, idx_v, ones_v, buf_v, deg_sh):
        c = lax.axis_index("c")
        s = lax.axis_index("s")
        wid = s * NC + c

        one = jnp.full((16,), 1.0, jnp.float32)
        for j in range(CHUNK // 16):
            ones_v[pl.ds(j * 16, 16)] = one
        zero = jnp.zeros((16,), jnp.float32)

        def zbody(i, _):
            buf_v[pl.ds(i * 16, 16)] = zero
            return ()
        lax.fori_loop(0, PPT // 16, zbody, ())
        pltpu.sync_copy(buf_v, deg_sh.at[pl.ds(s * PPT, PPT)])
        plsc.subcore_barrier()

        base = wid * EPT

        def body(i, _):
            pltpu.sync_copy(dst_hbm.at[pl.ds(base + i * CHUNK, CHUNK)], idx_v)
            pltpu.sync_copy(ones_v, deg_sh.at[idx_v], add=True)
            return ()
        lax.fori_loop(0, NITER, body, ())
        plsc.subcore_barrier()

        pltpu.sync_copy(deg_sh.at[pl.ds(s * PPT, PPT)], buf_v)
        pltpu.sync_copy(buf_v, out_hbm.at[c, pl.ds(s * PPT, PPT)])

    return deg_kernel


def _make_msg_kernel(E, N, D, NR):
    EPT = E // NW
    NITER = EPT // CHUNK
    RPT = NR // NS

    @functools.partial(
        pl.kernel,
        out_type=jax.ShapeDtypeStruct((NC, NR, D), jnp.float32),
        mesh=plsc.VectorSubcoreMesh(**_MESH),
        scratch_types=[
            pltpu.VMEM((CHUNK,), jnp.int32),
            pltpu.VMEM((CHUNK,), jnp.int32),
            pltpu.VMEM((CHUNK, D), jnp.float32),
            pltpu.VMEM_SHARED((NR, D), jnp.float32),
            pltpu.SemaphoreType.DMA,
        ],
    )
    def msg_kernel(src_hbm, dst_hbm, y_hbm, zeros_hbm, out_hbm,
                   sidx_v, didx_v, rows_v, s_sh, sem):
        c = lax.axis_index("c")
        s = lax.axis_index("s")
        wid = s * NC + c

        pltpu.sync_copy(zeros_hbm, s_sh.at[pl.ds(s * RPT, RPT)])
        plsc.subcore_barrier()

        base = wid * EPT

        def body(i, _):
            off = base + i * CHUNK
            pltpu.sync_copy(src_hbm.at[pl.ds(off, CHUNK)], sidx_v)
            pltpu.sync_copy(dst_hbm.at[pl.ds(off, CHUNK)], didx_v)
            pltpu.async_copy(y_hbm.at[sidx_v], rows_v, sem).wait()
            pltpu.sync_copy(rows_v, s_sh.at[didx_v], add=True)
            return ()
        lax.fori_loop(0, NITER, body, ())
        plsc.subcore_barrier()

        pltpu.sync_copy(s_sh.at[pl.ds(s * RPT, RPT)],
                        out_hbm.at[c, pl.ds(s * RPT, RPT)])

    return msg_kernel


def _scale_body(x_ref, wg_ref, dp0_ref, dp1_ref, y_ref, dinv_ref):
    deg = dp0_ref[...] + dp1_ref[...] + 1.0
    dinv = lax.rsqrt(deg)
    dinv_ref[...] = dinv
    xw = jnp.dot(x_ref[...], wg_ref[...], preferred_element_type=jnp.float32)
    y_ref[...] = xw * dinv


def _final_body(s_ref, y_ref, dinv_ref, x_ref, bg_ref,
                w1_ref, b1_ref, w2_ref, b2_ref, w3_ref, b3_ref, out_ref):
    n = y_ref.shape[0]
    agg = (dinv_ref[...] * (s_ref[0, :n] + s_ref[1, :n] + y_ref[...])
           + bg_ref[...])
    h = jnp.sum(jnp.maximum(agg, 0.0) + x_ref[...], axis=0, keepdims=True)
    h = jnp.maximum(
        jnp.dot(h, w1_ref[...], preferred_element_type=jnp.float32)
        + b1_ref[...], 0.0)
    h = jnp.maximum(
        jnp.dot(h, w2_ref[...], preferred_element_type=jnp.float32)
        + b2_ref[...], 0.0)
    out_ref[...] = (
        jnp.dot(h, w3_ref[...], preferred_element_type=jnp.float32)
        + b3_ref[...])


def kernel(x, edge_index, Wg, bg, W1, b1, W2, b2, W3, b3):
    N, D = x.shape
    E = edge_index.shape[1]
    NP = ((N + NS * 16 - 1) // (NS * 16)) * (NS * 16)

    src = edge_index[0]
    dst = edge_index[1]

    dp = _make_deg_kernel(E, NP)(dst)

    y, dinv = pl.pallas_call(
        _scale_body,
        out_shape=(
            jax.ShapeDtypeStruct((N, D), jnp.float32),
            jax.ShapeDtypeStruct((N, 1), jnp.float32),
        ),
    )(x, Wg, dp[0, :N, None], dp[1, :N, None])

    zeros = jnp.zeros((NP // NS, D), jnp.float32)
    S = _make_msg_kernel(E, N, D, NP)(src, dst, y, zeros)

    value = pl.pallas_call(
        _final_body,
        out_shape=jax.ShapeDtypeStruct((1, 1), jnp.float32),
    )(S, y, dinv, x, bg.reshape(1, D),
      W1, b1.reshape(1, -1), W2, b2.reshape(1, -1), W3, b3.reshape(1, -1))

    return value.reshape(1)
